# baseline (speedup 1.0000x reference)
import functools

import jax
import jax.numpy as jnp
from jax.experimental import pallas as pl
from jax.experimental.pallas import tpu as pltpu

NUM_EXPERTS = 8
TOP_K = 2
D_MODEL = 1024
D_FF = 4096
OUT_DIM = 7
N_TOK = 2048

LANES = 128
TM = 256
TF = 1024
NT = N_TOK // TM
NF = D_FF // TF


def _gelu(x):
    x3 = x * x * x
    return 0.5 * x * (1.0 + jnp.tanh(0.7978845608028654 * (x + 0.044715 * x3)))


def _gate_body(x_ref, wg_ref, bg_ref, meta_ref):
    x = x_ref[...]
    l = jnp.dot(x, wg_ref[...], preferred_element_type=jnp.float32)
    l = l + bg_ref[...]
    lane = jax.lax.broadcasted_iota(jnp.int32, l.shape, 1)
    neg = jnp.float32(-jnp.inf)
    l = jnp.where(lane < NUM_EXPERTS, l, neg)
    m1 = jnp.max(l, axis=1, keepdims=True)
    a1 = jnp.min(jnp.where(l == m1, lane, LANES), axis=1, keepdims=True)
    l2 = jnp.where(lane == a1, neg, l)
    m2 = jnp.max(l2, axis=1, keepdims=True)
    a2 = jnp.min(jnp.where(l2 == m2, lane, LANES), axis=1, keepdims=True)
    w1 = 1.0 / (1.0 + jnp.exp(m2 - m1))
    w2 = 1.0 - w1
    meta = (jnp.where(lane == 0, a1.astype(jnp.float32), 0.0)
            + jnp.where(lane == 1, a2.astype(jnp.float32), 0.0)
            + jnp.where(lane == 2, w1, 0.0)
            + jnp.where(lane == 3, w2, 0.0))
    meta_ref[...] = meta


def _gate(x, wg_pad, bg_pad):
    return pl.pallas_call(
        _gate_body,
        grid=(NT,),
        in_specs=[
            pl.BlockSpec((TM, D_MODEL), lambda t: (t, 0)),
            pl.BlockSpec((D_MODEL, LANES), lambda t: (0, 0)),
            pl.BlockSpec((1, LANES), lambda t: (0, 0)),
        ],
        out_specs=pl.BlockSpec((TM, LANES), lambda t: (t, 0)),
        out_shape=jax.ShapeDtypeStruct((N_TOK, LANES), jnp.float32),
    )(x, wg_pad, bg_pad)


def _dense_body(x_ref, meta_ref, lab_ref, w1_ref, b1_ref, w2_ref, b2_ref,
                out_ref):
    e = pl.program_id(1)
    f = pl.program_id(2)

    @pl.when(jnp.logical_and(e == 0, f == 0))
    def _():
        out_ref[...] = jnp.zeros_like(out_ref)

    meta = meta_ref[...]
    ef = jnp.float32(e)
    w_e = (meta[:, 2:3] * (meta[:, 0:1] == ef).astype(jnp.float32)
           + meta[:, 3:4] * (meta[:, 1:2] == ef).astype(jnp.float32))

    h = jnp.dot(x_ref[...], w1_ref[0], preferred_element_type=jnp.float32)
    h = _gelu(h + b1_ref[0])
    part = jnp.dot(h, w2_ref[0], preferred_element_type=jnp.float32)

    acc = w_e * part

    @pl.when(f == 0)
    def _():
        out_ref[...] += w_e * (b2_ref[0] + lab_ref[...])

    out_ref[...] += acc


def _dense(x, meta, lab_pad, W1, b1, W2_pad, b2_pad):
    return pl.pallas_call(
        _dense_body,
        grid=(NT, NUM_EXPERTS, NF),
        in_specs=[
            pl.BlockSpec((TM, D_MODEL), lambda t, e, f: (t, 0)),
            pl.BlockSpec((TM, LANES), lambda t, e, f: (t, 0)),
            pl.BlockSpec((TM, LANES), lambda t, e, f: (t, 0)),
            pl.BlockSpec((1, D_MODEL, TF), lambda t, e, f: (e, 0, f)),
            pl.BlockSpec((1, 1, TF), lambda t, e, f: (e, 0, f)),
            pl.BlockSpec((1, TF, LANES), lambda t, e, f: (e, f, 0)),
            pl.BlockSpec((1, 1, LANES), lambda t, e, f: (e, 0, 0)),
        ],
        out_specs=pl.BlockSpec((TM, LANES), lambda t, e, f: (t, 0)),
        out_shape=jax.ShapeDtypeStruct((N_TOK, LANES), jnp.float32),
    )(x, meta, lab_pad, W1, b1, W2_pad, b2_pad)


def kernel(inputs, label, Wg, bg, W1, b1, W2, b2):
    wg_pad = jnp.zeros((D_MODEL, LANES), jnp.float32).at[:, :NUM_EXPERTS].set(Wg)
    bg_pad = jnp.zeros((1, LANES), jnp.float32).at[0, :NUM_EXPERTS].set(bg)
    meta = _gate(inputs, wg_pad, bg_pad)

    lab_pad = jnp.zeros((N_TOK, LANES), jnp.float32).at[:, :OUT_DIM].set(label)
    W2_pad = jnp.zeros((NUM_EXPERTS, D_FF, LANES), jnp.float32).at[:, :, :OUT_DIM].set(W2)
    b2_pad = jnp.zeros((NUM_EXPERTS, 1, LANES), jnp.float32).at[:, 0, :OUT_DIM].set(b2)
    b1_r = b1.reshape(NUM_EXPERTS, 1, D_FF)
    out = _dense(inputs, meta, lab_pad, W1, b1_r, W2_pad, b2_pad)
    return out[:, :OUT_DIM]


# speedup vs baseline: 2.3068x; 2.3068x over previous
import functools

import jax
import jax.numpy as jnp
from jax import lax
from jax.experimental import pallas as pl
from jax.experimental.pallas import tpu as pltpu
from jax.experimental.pallas import tpu_sc as plsc

NUM_EXPERTS = 8
TOP_K = 2
D_MODEL = 1024
D_FF = 4096
OUT_DIM = 7
N_TOK = 2048

LANES = 128
TF = 1024
NF = D_FF // TF
TMG = 128
NTG = 40
R = NTG * TMG
NARROW = 16


def _gelu(x):
    x3 = x * x * x
    return 0.5 * x * (1.0 + jnp.tanh(0.7978845608028654 * (x + 0.044715 * x3)))


def _route_body(x_ref, wg_ref, bg_ref, r8_ref, te_ref, s1_ref, s2_ref):
    l = jnp.dot(x_ref[...], wg_ref[...], preferred_element_type=jnp.float32)
    l = l + bg_ref[...]
    lane = lax.broadcasted_iota(jnp.int32, l.shape, 1)
    neg = jnp.float32(-jnp.inf)
    l = jnp.where(lane < NUM_EXPERTS, l, neg)
    m1 = jnp.max(l, axis=1, keepdims=True)
    a1 = jnp.min(jnp.where(l == m1, lane, LANES), axis=1, keepdims=True)
    l2 = jnp.where(lane == a1, neg, l)
    m2 = jnp.max(l2, axis=1, keepdims=True)
    a2 = jnp.min(jnp.where(l2 == m2, lane, LANES), axis=1, keepdims=True)
    w1 = 1.0 / (1.0 + jnp.exp(m2 - m1))
    w2 = 1.0 - w1

    lane8 = lax.broadcasted_iota(jnp.int32, (N_TOK, NUM_EXPERTS), 1)
    one1 = (lane8 == a1).astype(jnp.float32)
    one2 = (lane8 == a2).astype(jnp.float32)

    tl = (lax.broadcasted_iota(jnp.int32, (128, 128), 0)
          > lax.broadcasted_iota(jnp.int32, (128, 128), 1)).astype(jnp.float32)
    c1 = jnp.zeros((1, NUM_EXPERTS), jnp.float32)
    c2 = jnp.zeros((1, NUM_EXPERTS), jnp.float32)
    for b in range(N_TOK // 128):
        sl = pl.ds(b * 128, 128)
        o1b = one1[b * 128:(b + 1) * 128, :]
        o2b = one2[b * 128:(b + 1) * 128, :]
        s1_ref[sl, :] = jnp.dot(tl, o1b, preferred_element_type=jnp.float32) + c1
        s2_ref[sl, :] = jnp.dot(tl, o2b, preferred_element_type=jnp.float32) + c2
        c1 = c1 + jnp.sum(o1b, axis=0, keepdims=True)
        c2 = c2 + jnp.sum(o2b, axis=0, keepdims=True)

    counts = c1 + c2
    padc = jnp.floor((counts + 127.0) * (1.0 / 128.0)) * 128.0
    su = (lax.broadcasted_iota(jnp.int32, (NUM_EXPERTS, NUM_EXPERTS), 0)
          < lax.broadcasted_iota(jnp.int32, (NUM_EXPERTS, NUM_EXPERTS), 1)
          ).astype(jnp.float32)
    offs = jnp.dot(padc, su, preferred_element_type=jnp.float32)
    ends = offs + padc

    rp1 = jnp.sum(one1 * (offs + s1_ref[...]), axis=1, keepdims=True)
    rp2 = jnp.sum(one2 * (offs + c1 + s2_ref[...]), axis=1, keepdims=True)

    r8 = (jnp.where(lane8 == 0, rp1, 0.0)
          + jnp.where(lane8 == 1, rp2, 0.0)
          + jnp.where(lane8 == 2, w1, 0.0)
          + jnp.where(lane8 == 3, w2, 0.0))
    r8_ref[...] = r8

    i64 = lax.broadcasted_iota(jnp.int32, (8, 64), 1).astype(jnp.float32) * 128.0
    te = jnp.zeros((8, 64), jnp.float32)
    for e in range(NUM_EXPERTS):
        te = te + (i64 >= ends[0, e]).astype(jnp.float32)
    te_ref[...] = jnp.minimum(te, NUM_EXPERTS - 1)


def _route(x, wg_pad, bg_pad):
    return pl.pallas_call(
        _route_body,
        grid=(1,),
        in_specs=[
            pl.BlockSpec((N_TOK, D_MODEL), lambda i: (0, 0)),
            pl.BlockSpec((D_MODEL, LANES), lambda i: (0, 0)),
            pl.BlockSpec((1, LANES), lambda i: (0, 0)),
        ],
        out_specs=[
            pl.BlockSpec((N_TOK, NUM_EXPERTS), lambda i: (0, 0)),
            pl.BlockSpec((8, 64), lambda i: (0, 0)),
        ],
        out_shape=[
            jax.ShapeDtypeStruct((N_TOK, NUM_EXPERTS), jnp.float32),
            jax.ShapeDtypeStruct((8, 64), jnp.float32),
        ],
        scratch_shapes=[
            pltpu.VMEM((N_TOK, NUM_EXPERTS), jnp.float32),
            pltpu.VMEM((N_TOK, NUM_EXPERTS), jnp.float32),
        ],
    )(x, wg_pad, bg_pad)


def _sc_mesh():
    return plsc.VectorSubcoreMesh(core_axis_name="c", subcore_axis_name="s")


def _dispatch(x, rp1, rp2):
    per_w = N_TOK // 32

    @functools.partial(
        pl.kernel,
        mesh=_sc_mesh(),
        out_type=jax.ShapeDtypeStruct((R, D_MODEL), jnp.float32),
        scratch_types=[
            pltpu.VMEM((per_w,), jnp.int32),
            pltpu.VMEM((per_w,), jnp.int32),
            pltpu.VMEM((per_w, D_MODEL), jnp.float32),
            pltpu.SemaphoreType.DMA,
            pltpu.SemaphoreType.DMA,
        ],
    )
    def dispatch_k(x_hbm, rp1_hbm, rp2_hbm, xs_hbm, i1b, i2b, xb, semA, semB):
        c = lax.axis_index("c")
        s = lax.axis_index("s")
        wid = s * 2 + c
        base = pl.multiple_of(wid * per_w, 64)
        pltpu.sync_copy(rp1_hbm.at[pl.ds(base, per_w)], i1b)
        pltpu.sync_copy(rp2_hbm.at[pl.ds(base, per_w)], i2b)
        pltpu.sync_copy(x_hbm.at[pl.ds(base, per_w)], xb)
        cp1 = pltpu.async_copy(xb, xs_hbm.at[i1b], semA)
        cp2 = pltpu.async_copy(xb, xs_hbm.at[i2b], semB)
        cp1.wait()
        cp2.wait()

    return dispatch_k(x, rp1, rp2)


def _group_body(te_ref, xs_ref, w1_ref, b1_ref, w2_ref, b2_ref, ys_ref, h_ref):
    i = pl.program_id(0)
    e = te_ref[i]
    x = xs_ref[...].astype(jnp.bfloat16)
    for f in range(NF):
        fsl = pl.ds(f * TF, TF)
        h = jnp.dot(x, w1_ref[0, :, fsl], preferred_element_type=jnp.float32)
        h_ref[:, fsl] = _gelu(h + b1_ref[e, 0, fsl]).astype(jnp.bfloat16)
    acc = jnp.zeros((TMG, LANES), jnp.float32)
    for f in range(NF):
        fsl = pl.ds(f * TF, TF)
        acc = acc + jnp.dot(h_ref[:, fsl], w2_ref[e, fsl, :],
                            preferred_element_type=jnp.float32)
    ys_ref[...] = acc + b2_ref[e, 0, :]


def _group(te, xs, W1_bf, b1_r, W2_bf, b2_pad):
    grid_spec = pltpu.PrefetchScalarGridSpec(
        num_scalar_prefetch=1,
        grid=(NTG,),
        in_specs=[
            pl.BlockSpec((TMG, D_MODEL), lambda i, te: (i, 0)),
            pl.BlockSpec((1, D_MODEL, D_FF), lambda i, te: (te[i], 0, 0)),
            pl.BlockSpec((NUM_EXPERTS, 1, D_FF), lambda i, te: (0, 0, 0)),
            pl.BlockSpec((NUM_EXPERTS, D_FF, LANES), lambda i, te: (0, 0, 0)),
            pl.BlockSpec((NUM_EXPERTS, 1, LANES), lambda i, te: (0, 0, 0)),
        ],
        out_specs=pl.BlockSpec((TMG, LANES), lambda i, te: (i, 0)),
        scratch_shapes=[pltpu.VMEM((TMG, D_FF), jnp.bfloat16)],
    )
    return pl.pallas_call(
        _group_body,
        grid_spec=grid_spec,
        out_shape=jax.ShapeDtypeStruct((R, LANES), jnp.float32),
    )(te, xs, W1_bf, b1_r, W2_bf, b2_pad)


def _combine(rp1, rp2, w1v, w2v, ys, lab16):
    per_w = N_TOK // 32

    @functools.partial(
        pl.kernel,
        mesh=_sc_mesh(),
        out_type=jax.ShapeDtypeStruct((N_TOK, NARROW), jnp.float32),
        scratch_types=[
            pltpu.VMEM((per_w,), jnp.int32),
            pltpu.VMEM((per_w,), jnp.int32),
            pltpu.VMEM((per_w,), jnp.float32),
            pltpu.VMEM((per_w,), jnp.float32),
            pltpu.VMEM((per_w, NARROW), jnp.float32),
            pltpu.VMEM((per_w, LANES), jnp.float32),
            pltpu.VMEM((per_w, LANES), jnp.float32),
            pltpu.VMEM((per_w, NARROW), jnp.float32),
            pltpu.SemaphoreType.DMA,
        ],
    )
    def combine_k(rp1_hbm, rp2_hbm, w1_hbm, w2_hbm, ys_hbm, lab_hbm, out_hbm,
                  i1b, i2b, w1b, w2b, lbuf, y1, y2, obuf, sem):
        c = lax.axis_index("c")
        s = lax.axis_index("s")
        wid = s * 2 + c
        base = pl.multiple_of(wid * per_w, 64)
        pltpu.sync_copy(rp1_hbm.at[pl.ds(base, per_w)], i1b)
        pltpu.sync_copy(rp2_hbm.at[pl.ds(base, per_w)], i2b)
        pltpu.sync_copy(w1_hbm.at[pl.ds(base, per_w)], w1b)
        pltpu.sync_copy(w2_hbm.at[pl.ds(base, per_w)], w2b)
        pltpu.sync_copy(lab_hbm.at[pl.ds(base, per_w)], lbuf)
        cp1 = pltpu.async_copy(ys_hbm.at[i1b], y1, sem)
        cp2 = pltpu.async_copy(ys_hbm.at[i2b], y2, sem)
        cp1.wait()
        cp2.wait()
        for g in range(per_w // 16):
            w1g = w1b[pl.ds(g * 16, 16)]
            w2g = w2b[pl.ds(g * 16, 16)]
            for k in range(16):
                j = g * 16 + k
                obuf[j] = (w1g[k] * y1[j, 0:NARROW] + w2g[k] * y2[j, 0:NARROW]
                           + lbuf[j])
        pltpu.sync_copy(obuf, out_hbm.at[pl.ds(base, per_w)])

    return combine_k(rp1, rp2, w1v, w2v, ys, lab16)


def kernel(inputs, label, Wg, bg, W1, b1, W2, b2):
    wg_pad = jnp.zeros((D_MODEL, LANES), jnp.float32).at[:, :NUM_EXPERTS].set(Wg)
    bg_pad = jnp.zeros((1, LANES), jnp.float32).at[0, :NUM_EXPERTS].set(bg)
    r8, te8 = _route(inputs, wg_pad, bg_pad)
    rp1 = r8[:, 0].astype(jnp.int32)
    rp2 = r8[:, 1].astype(jnp.int32)
    w1v = r8[:, 2]
    w2v = r8[:, 3]
    te = te8[0, :NTG].astype(jnp.int32)

    xs = _dispatch(inputs, rp1, rp2)

    W1_bf = W1.astype(jnp.bfloat16)
    W2_bf = jnp.zeros((NUM_EXPERTS, D_FF, LANES), jnp.bfloat16).at[
        :, :, :OUT_DIM].set(W2.astype(jnp.bfloat16))
    b2_pad = jnp.zeros((NUM_EXPERTS, 1, LANES), jnp.float32).at[
        :, 0, :OUT_DIM].set(b2)
    b1_r = b1.reshape(NUM_EXPERTS, 1, D_FF)
    ys = _group(te, xs, W1_bf, b1_r, W2_bf, b2_pad)

    lab16 = jnp.zeros((N_TOK, NARROW), jnp.float32).at[:, :OUT_DIM].set(label)
    out = _combine(rp1, rp2, w1v, w2v, ys, lab16)
    return out[:, :OUT_DIM]
